# baseline (speedup 1.0000x reference)
import functools

import jax
import jax.numpy as jnp
from jax import lax
from jax.experimental import pallas as pl
from jax.experimental.pallas import tpu as pltpu
from jax.experimental.pallas import tpu_sc as plsc

K = 1024
D = 64
DA = 72
C = 256
N = 64 * 576
T = 512
LANES = 128

CHUNKS = (16384, 12288, 8192)
OFFSETS = (0, 16384, 28672)
FULL = 0
CH = len(CHUNKS)

LN2 = 0.6931471805599453
KF = float(K)


def _table_kernel(cb_ref, wr_ref, br_ref, tab_ref):
    tab_ref[...] = jnp.dot(cb_ref[...], wr_ref[...],
                           preferred_element_type=jnp.float32) + br_ref[...]


def _chunk_kernel(nsteps, x_ref, ws_ref, bs_ref, cb_ref,
                  codes_ref, loss_ref, cnt_ref,
                  m_acc, sqr_acc, ids_acc, loss_acc, cnt_acc):
    i = pl.program_id(0)

    @pl.when(i == 0)
    def _init():
        cb = cb_ref[...]
        m_acc[...] = cb * -2.0
        sqr_acc[...] = jnp.sum(cb * cb, axis=1, keepdims=True)
        ids_acc[...] = lax.broadcasted_iota(
            jnp.int32, (K, T), 0).astype(jnp.float32)
        loss_acc[0, 0] = 0.0
        cnt_acc[...] = jnp.zeros_like(cnt_acc)

    x = x_ref[...]
    z = jnp.dot(x, ws_ref[...],
                preferred_element_type=jnp.float32) + bs_ref[...]
    zt = z.T
    scores = sqr_acc[...] + jnp.dot(
        m_acc[...], zt, preferred_element_type=jnp.float32)
    minv = jnp.min(scores, axis=0, keepdims=True)
    mask = scores == minv
    codes_row = jnp.min(jnp.where(mask, ids_acc[...], KF), axis=0)
    codes_ref[0, 0, :] = codes_row.astype(jnp.int32)
    loss_acc[0, 0] += jnp.sum(minv) + jnp.sum(z * z)
    maskf = jnp.where(mask, 1.0, 0.0)
    cnt_acc[...] += (maskf[:, 0:LANES] + maskf[:, LANES:2 * LANES]
                     + maskf[:, 2 * LANES:3 * LANES]
                     + maskf[:, 3 * LANES:4 * LANES])

    @pl.when(i == nsteps - 1)
    def _fin():
        loss_ref[0, 0] = loss_acc[0, 0]
        cnt_ref[...] = cnt_acc[...]


def _fin_kernel(l_ref, c_ref, loss_ref, ent_ref):
    loss_ref[0, 0] = jnp.sum(l_ref[...])
    cnt = jnp.sum(c_ref[...], axis=1, keepdims=True)
    p = cnt * (1.0 / N)
    plogp = jnp.where(p > 0.0, p * jnp.log(p), 0.0)
    ent_ref[0, 0] = -jnp.sum(plogp) * (1.0 / LN2)


def _chunk_call(c, x2, W_send, b_send, codebook):
    nsteps = CHUNKS[c] // T
    step0 = OFFSETS[c] // T
    return pl.pallas_call(
        functools.partial(_chunk_kernel, nsteps),
        grid=(nsteps,),
        in_specs=[
            pl.BlockSpec((T, C), lambda i, s=step0: (s + i, 0)),
            pl.BlockSpec((C, D), lambda i: (0, 0)),
            pl.BlockSpec((1, D), lambda i: (0, 0)),
            pl.BlockSpec((K, D), lambda i: (0, 0)),
        ],
        out_specs=[
            pl.BlockSpec((1, 1, T), lambda i: (i, 0, 0)),
            pl.BlockSpec(memory_space=pltpu.SMEM),
            pl.BlockSpec((K, LANES), lambda i: (0, 0)),
        ],
        out_shape=[
            jax.ShapeDtypeStruct((nsteps, 1, T), jnp.int32),
            jax.ShapeDtypeStruct((1, 1), jnp.float32),
            jax.ShapeDtypeStruct((K, LANES), jnp.float32),
        ],
        scratch_shapes=[
            pltpu.VMEM((K, D), jnp.float32),
            pltpu.VMEM((K, 1), jnp.float32),
            pltpu.VMEM((K, T), jnp.float32),
            pltpu.SMEM((1, 1), jnp.float32),
            pltpu.VMEM((K, LANES), jnp.float32),
        ],
    )(x2, W_send, b_send, codebook)


GW = 128


def _sc_gather(table, idx2, out_rows, row0=0):
    nidx = idx2.shape[1]
    blk0 = row0 // GW
    mesh = plsc.VectorSubcoreMesh(core_axis_name="core",
                                  subcore_axis_name="subcore")

    @functools.partial(
        pl.kernel,
        out_type=jax.ShapeDtypeStruct((out_rows, C), jnp.float32),
        mesh=mesh,
    )
    def k(tab_hbm, idx_hbm, out_hbm):
        def body(idx_vmem, out_vmem):
            pltpu.sync_copy(tab_hbm.at[idx_vmem.at[0]], out_vmem)

        pltpu.emit_pipeline(
            body,
            grid=(nidx // GW,),
            in_specs=[pl.BlockSpec((1, GW), index_map=lambda i: (0, i))],
            out_specs=[pl.BlockSpec((GW, C),
                                    index_map=lambda i: (blk0 + i, 0))],
            core_axis_name=("core", "subcore"),
            dimension_semantics=(pltpu.PARALLEL,),
        )(idx_hbm, out_hbm)

    return k(table, idx2)


def kernel(input, W_send, b_send, codebook, W_recv, b_recv):
    x2 = input.reshape(N, C)

    table = pl.pallas_call(
        _table_kernel,
        out_shape=jax.ShapeDtypeStruct((K, C), jnp.float32),
    )(codebook, W_recv, b_recv.reshape(1, C))

    codes_l, loss_l, cnt_l, x_l = [], [], [], []
    for c in range(CH):
        codes3, loss_c, cnt_c = _chunk_call(
            c, x2, W_send, b_send.reshape(1, D), codebook)
        codes_l.append(codes3.reshape(1, CHUNKS[c]))
        loss_l.append(loss_c)
        cnt_l.append(cnt_c)
        if c == FULL:
            x_l.append(_sc_gather(table, codes_l[c], N, row0=OFFSETS[c]))
        else:
            x_l.append(_sc_gather(table, codes_l[c], CHUNKS[c]))

    x = x_l[FULL]
    for c in range(CH):
        if c != FULL:
            x = lax.dynamic_update_slice(x, x_l[c], (OFFSETS[c], 0))

    loss, ent = pl.pallas_call(
        _fin_kernel,
        in_specs=[
            pl.BlockSpec((1, CH), lambda: (0, 0)),
            pl.BlockSpec((K, CH * LANES), lambda: (0, 0)),
        ],
        out_specs=[
            pl.BlockSpec(memory_space=pltpu.SMEM),
            pl.BlockSpec(memory_space=pltpu.SMEM),
        ],
        out_shape=[
            jax.ShapeDtypeStruct((1, 1), jnp.float32),
            jax.ShapeDtypeStruct((1, 1), jnp.float32),
        ],
    )(jnp.concatenate(loss_l, axis=1),
      jnp.concatenate(cnt_l, axis=1))

    codes = jnp.concatenate(codes_l, axis=1).reshape(64, 576)
    loss0 = loss.reshape(())
    return (x.reshape(64, 576, C), codes, loss0, loss0, ent.reshape(()))


# speedup vs baseline: 1.0457x; 1.0457x over previous
import functools

import jax
import jax.numpy as jnp
from jax import lax
from jax.experimental import pallas as pl
from jax.experimental.pallas import tpu as pltpu
from jax.experimental.pallas import tpu_sc as plsc

K = 1024
D = 64
DA = 72
C = 256
N = 64 * 576
T = 512
LANES = 128

CHUNKS = (16384, 12288, 8192)
OFFSETS = (0, 16384, 28672)
FULL = 0
CH = len(CHUNKS)

LN2 = 0.6931471805599453
KF = float(K)


def _table_kernel(cb_ref, wr_ref, br_ref, tab_ref):
    tab_ref[...] = jnp.dot(cb_ref[...], wr_ref[...],
                           preferred_element_type=jnp.float32) + br_ref[...]


def _chunk_kernel(nsteps, x_ref, ws_ref, bs_ref, cbt_ref,
                  codes_ref, loss_ref, cnt_ref,
                  sqr_acc, cbt2_acc, ids_acc, loss_acc, cnt_acc):
    i = pl.program_id(0)

    @pl.when(i == 0)
    def _init():
        cbt = cbt_ref[...]
        sqr_acc[...] = jnp.sum(cbt * cbt, axis=0, keepdims=True)
        cbt2_acc[...] = cbt * -2.0
        ids_acc[...] = lax.broadcasted_iota(
            jnp.int32, (1, K), 1).astype(jnp.float32)
        loss_acc[0, 0] = 0.0
        cnt_acc[...] = jnp.zeros_like(cnt_acc)

    x = x_ref[...]
    z = jnp.dot(x, ws_ref[...],
                preferred_element_type=jnp.float32) + bs_ref[...]
    scores = sqr_acc[...] + jnp.dot(
        z, cbt2_acc[...], preferred_element_type=jnp.float32)
    minv = jnp.min(scores, axis=-1, keepdims=True)
    mask = scores == minv
    codes_f = jnp.min(jnp.where(mask, ids_acc[...], KF), axis=-1)
    codes_ref[0, 0, :] = codes_f.astype(jnp.int32)
    loss_acc[0, 0] += jnp.sum(minv) + jnp.sum(z * z)
    maskf = jnp.where(mask, 1.0, 0.0)
    cnt_acc[...] += jnp.sum(maskf, axis=0, keepdims=True)

    @pl.when(i == nsteps - 1)
    def _fin():
        loss_ref[0, 0] = loss_acc[0, 0]
        cnt_ref[...] = cnt_acc[...]


def _fin_kernel(l_ref, c_ref, loss_ref, ent_ref):
    loss_ref[0, 0] = jnp.sum(l_ref[...])
    cnt = jnp.sum(c_ref[...], axis=0, keepdims=True)
    p = cnt * (1.0 / N)
    plogp = jnp.where(p > 0.0, p * jnp.log(p), 0.0)
    ent_ref[0, 0] = -jnp.sum(plogp) * (1.0 / LN2)


def _chunk_call(c, x2, W_send, b_send, cbT):
    nsteps = CHUNKS[c] // T
    step0 = OFFSETS[c] // T
    return pl.pallas_call(
        functools.partial(_chunk_kernel, nsteps),
        grid=(nsteps,),
        in_specs=[
            pl.BlockSpec((T, C), lambda i, s=step0: (s + i, 0)),
            pl.BlockSpec((C, D), lambda i: (0, 0)),
            pl.BlockSpec((1, D), lambda i: (0, 0)),
            pl.BlockSpec((D, K), lambda i: (0, 0)),
        ],
        out_specs=[
            pl.BlockSpec((1, 1, T), lambda i: (i, 0, 0)),
            pl.BlockSpec(memory_space=pltpu.SMEM),
            pl.BlockSpec((1, K), lambda i: (0, 0)),
        ],
        out_shape=[
            jax.ShapeDtypeStruct((nsteps, 1, T), jnp.int32),
            jax.ShapeDtypeStruct((1, 1), jnp.float32),
            jax.ShapeDtypeStruct((1, K), jnp.float32),
        ],
        scratch_shapes=[
            pltpu.VMEM((1, K), jnp.float32),
            pltpu.VMEM((D, K), jnp.float32),
            pltpu.VMEM((1, K), jnp.float32),
            pltpu.SMEM((1, 1), jnp.float32),
            pltpu.VMEM((1, K), jnp.float32),
        ],
    )(x2, W_send, b_send, cbT)


GW = 128


def _sc_gather(table, idx2, out_rows, row0=0):
    nidx = idx2.shape[1]
    blk0 = row0 // GW
    mesh = plsc.VectorSubcoreMesh(core_axis_name="core",
                                  subcore_axis_name="subcore")

    @functools.partial(
        pl.kernel,
        out_type=jax.ShapeDtypeStruct((out_rows, C), jnp.float32),
        mesh=mesh,
    )
    def k(tab_hbm, idx_hbm, out_hbm):
        def body(idx_vmem, out_vmem):
            pltpu.sync_copy(tab_hbm.at[idx_vmem.at[0]], out_vmem)

        pltpu.emit_pipeline(
            body,
            grid=(nidx // GW,),
            in_specs=[pl.BlockSpec((1, GW), index_map=lambda i: (0, i))],
            out_specs=[pl.BlockSpec((GW, C),
                                    index_map=lambda i: (blk0 + i, 0))],
            core_axis_name=("core", "subcore"),
            dimension_semantics=(pltpu.PARALLEL,),
        )(idx_hbm, out_hbm)

    return k(table, idx2)


def kernel(input, W_send, b_send, codebook, W_recv, b_recv):
    x2 = input.reshape(N, C)
    cbT = codebook.T

    table = pl.pallas_call(
        _table_kernel,
        out_shape=jax.ShapeDtypeStruct((K, C), jnp.float32),
    )(codebook, W_recv, b_recv.reshape(1, C))

    codes_l, loss_l, cnt_l, x_l = [], [], [], []
    for c in range(CH):
        codes3, loss_c, cnt_c = _chunk_call(
            c, x2, W_send, b_send.reshape(1, D), cbT)
        codes_l.append(codes3.reshape(1, CHUNKS[c]))
        loss_l.append(loss_c)
        cnt_l.append(cnt_c)
        if c == FULL:
            x_l.append(_sc_gather(table, codes_l[c], N, row0=OFFSETS[c]))
        else:
            x_l.append(_sc_gather(table, codes_l[c], CHUNKS[c]))

    x = x_l[FULL]
    for c in range(CH):
        if c != FULL:
            x = lax.dynamic_update_slice(x, x_l[c], (OFFSETS[c], 0))

    loss, ent = pl.pallas_call(
        _fin_kernel,
        in_specs=[
            pl.BlockSpec((1, CH), lambda: (0, 0)),
            pl.BlockSpec((CH, K), lambda: (0, 0)),
        ],
        out_specs=[
            pl.BlockSpec(memory_space=pltpu.SMEM),
            pl.BlockSpec(memory_space=pltpu.SMEM),
        ],
        out_shape=[
            jax.ShapeDtypeStruct((1, 1), jnp.float32),
            jax.ShapeDtypeStruct((1, 1), jnp.float32),
        ],
    )(jnp.concatenate(loss_l, axis=1),
      jnp.concatenate(cnt_l, axis=0))

    codes = jnp.concatenate(codes_l, axis=1).reshape(64, 576)
    loss0 = loss.reshape(())
    return (x.reshape(64, 576, C), codes, loss0, loss0, ent.reshape(()))


# speedup vs baseline: 1.0820x; 1.0347x over previous
import functools

import jax
import jax.numpy as jnp
from jax import lax
from jax.experimental import pallas as pl
from jax.experimental.pallas import tpu as pltpu
from jax.experimental.pallas import tpu_sc as plsc

K = 1024
D = 64
DA = 72
C = 256
N = 64 * 576
T = 1024
LANES = 128

CHUNKS = (16384, 12288, 8192)
OFFSETS = (0, 16384, 28672)
FULL = 0
CH = len(CHUNKS)

LN2 = 0.6931471805599453
KF = float(K)


def _table_kernel(cb_ref, wr_ref, br_ref, tab_ref):
    tab_ref[...] = jnp.dot(cb_ref[...], wr_ref[...],
                           preferred_element_type=jnp.float32) + br_ref[...]


def _chunk_kernel(nsteps, x_ref, ws_ref, bs_ref, cbt_ref,
                  codes_ref, loss_ref, cnt_ref,
                  sqr_acc, cbt2_acc, ids_acc, loss_acc, cnt_acc):
    i = pl.program_id(0)

    @pl.when(i == 0)
    def _init():
        cbt = cbt_ref[...]
        sqr_acc[...] = jnp.sum(cbt * cbt, axis=0, keepdims=True)
        cbt2_acc[...] = cbt * -2.0
        ids_acc[...] = lax.broadcasted_iota(
            jnp.int32, (1, K), 1).astype(jnp.float32)
        loss_acc[0, 0] = 0.0
        cnt_acc[...] = jnp.zeros_like(cnt_acc)

    x = x_ref[...]
    z = jnp.dot(x, ws_ref[...],
                preferred_element_type=jnp.float32) + bs_ref[...]
    scores = sqr_acc[...] + jnp.dot(
        z, cbt2_acc[...], preferred_element_type=jnp.float32)
    minv = jnp.min(scores, axis=-1, keepdims=True)
    mask = scores == minv
    codes_f = jnp.min(jnp.where(mask, ids_acc[...], KF), axis=-1)
    codes_ref[0, 0, :] = codes_f.astype(jnp.int32)
    loss_acc[0, 0] += jnp.sum(minv) + jnp.sum(z * z)
    maskf = jnp.where(mask, 1.0, 0.0)
    cnt_acc[...] += jnp.sum(maskf, axis=0, keepdims=True)

    @pl.when(i == nsteps - 1)
    def _fin():
        loss_ref[0, 0] = loss_acc[0, 0]
        cnt_ref[...] = cnt_acc[...]


def _fin_kernel(l_ref, c_ref, loss_ref, ent_ref):
    loss_ref[0, 0] = jnp.sum(l_ref[...])
    cnt = jnp.sum(c_ref[...], axis=0, keepdims=True)
    p = cnt * (1.0 / N)
    plogp = jnp.where(p > 0.0, p * jnp.log(p), 0.0)
    ent_ref[0, 0] = -jnp.sum(plogp) * (1.0 / LN2)


def _chunk_call(c, x2, W_send, b_send, cbT):
    nsteps = CHUNKS[c] // T
    step0 = OFFSETS[c] // T
    return pl.pallas_call(
        functools.partial(_chunk_kernel, nsteps),
        grid=(nsteps,),
        in_specs=[
            pl.BlockSpec((T, C), lambda i, s=step0: (s + i, 0)),
            pl.BlockSpec((C, D), lambda i: (0, 0)),
            pl.BlockSpec((1, D), lambda i: (0, 0)),
            pl.BlockSpec((D, K), lambda i: (0, 0)),
        ],
        out_specs=[
            pl.BlockSpec((1, 1, T), lambda i: (i, 0, 0)),
            pl.BlockSpec(memory_space=pltpu.SMEM),
            pl.BlockSpec((1, K), lambda i: (0, 0)),
        ],
        out_shape=[
            jax.ShapeDtypeStruct((nsteps, 1, T), jnp.int32),
            jax.ShapeDtypeStruct((1, 1), jnp.float32),
            jax.ShapeDtypeStruct((1, K), jnp.float32),
        ],
        scratch_shapes=[
            pltpu.VMEM((1, K), jnp.float32),
            pltpu.VMEM((D, K), jnp.float32),
            pltpu.VMEM((1, K), jnp.float32),
            pltpu.SMEM((1, 1), jnp.float32),
            pltpu.VMEM((1, K), jnp.float32),
        ],
    )(x2, W_send, b_send, cbT)


GW = 128


def _sc_gather(table, idx2, out_rows, row0=0):
    nidx = idx2.shape[1]
    blk0 = row0 // GW
    mesh = plsc.VectorSubcoreMesh(core_axis_name="core",
                                  subcore_axis_name="subcore")

    @functools.partial(
        pl.kernel,
        out_type=jax.ShapeDtypeStruct((out_rows, C), jnp.float32),
        mesh=mesh,
    )
    def k(tab_hbm, idx_hbm, out_hbm):
        def body(idx_vmem, out_vmem):
            pltpu.sync_copy(tab_hbm.at[idx_vmem.at[0]], out_vmem)

        pltpu.emit_pipeline(
            body,
            grid=(nidx // GW,),
            in_specs=[pl.BlockSpec((1, GW), index_map=lambda i: (0, i))],
            out_specs=[pl.BlockSpec((GW, C),
                                    index_map=lambda i: (blk0 + i, 0))],
            core_axis_name=("core", "subcore"),
            dimension_semantics=(pltpu.PARALLEL,),
        )(idx_hbm, out_hbm)

    return k(table, idx2)


def kernel(input, W_send, b_send, codebook, W_recv, b_recv):
    x2 = input.reshape(N, C)
    cbT = codebook.T

    table = pl.pallas_call(
        _table_kernel,
        out_shape=jax.ShapeDtypeStruct((K, C), jnp.float32),
    )(codebook, W_recv, b_recv.reshape(1, C))

    codes_l, loss_l, cnt_l, x_l = [], [], [], []
    for c in range(CH):
        codes3, loss_c, cnt_c = _chunk_call(
            c, x2, W_send, b_send.reshape(1, D), cbT)
        codes_l.append(codes3.reshape(1, CHUNKS[c]))
        loss_l.append(loss_c)
        cnt_l.append(cnt_c)
        if c == FULL:
            x_l.append(_sc_gather(table, codes_l[c], N, row0=OFFSETS[c]))
        else:
            x_l.append(_sc_gather(table, codes_l[c], CHUNKS[c]))

    x = x_l[FULL]
    for c in range(CH):
        if c != FULL:
            x = lax.dynamic_update_slice(x, x_l[c], (OFFSETS[c], 0))

    loss, ent = pl.pallas_call(
        _fin_kernel,
        in_specs=[
            pl.BlockSpec((1, CH), lambda: (0, 0)),
            pl.BlockSpec((CH, K), lambda: (0, 0)),
        ],
        out_specs=[
            pl.BlockSpec(memory_space=pltpu.SMEM),
            pl.BlockSpec(memory_space=pltpu.SMEM),
        ],
        out_shape=[
            jax.ShapeDtypeStruct((1, 1), jnp.float32),
            jax.ShapeDtypeStruct((1, 1), jnp.float32),
        ],
    )(jnp.concatenate(loss_l, axis=1),
      jnp.concatenate(cnt_l, axis=0))

    codes = jnp.concatenate(codes_l, axis=1).reshape(64, 576)
    loss0 = loss.reshape(())
    return (x.reshape(64, 576, C), codes, loss0, loss0, ent.reshape(()))


# speedup vs baseline: 1.0923x; 1.0095x over previous
import functools

import jax
import jax.numpy as jnp
from jax import lax
from jax.experimental import pallas as pl
from jax.experimental.pallas import tpu as pltpu
from jax.experimental.pallas import tpu_sc as plsc

K = 1024
D = 64
DA = 72
C = 256
N = 64 * 576
T = 2048
LANES = 128

CHUNKS = (16384, 12288, 8192)
OFFSETS = (0, 16384, 28672)
FULL = 0
CH = len(CHUNKS)

LN2 = 0.6931471805599453
KF = float(K)


def _table_kernel(cb_ref, wr_ref, br_ref, tab_ref):
    tab_ref[...] = jnp.dot(cb_ref[...], wr_ref[...],
                           preferred_element_type=jnp.float32) + br_ref[...]


def _chunk_kernel(nsteps, x_ref, ws_ref, bs_ref, cbt_ref,
                  codes_ref, loss_ref, cnt_ref,
                  sqr_acc, cbt2_acc, ids_acc, loss_acc, cnt_acc):
    i = pl.program_id(0)

    @pl.when(i == 0)
    def _init():
        cbt = cbt_ref[...]
        sqr_acc[...] = jnp.sum(cbt * cbt, axis=0, keepdims=True)
        cbt2_acc[...] = cbt * -2.0
        ids_acc[...] = lax.broadcasted_iota(
            jnp.int32, (1, K), 1).astype(jnp.float32)
        loss_acc[0, 0] = 0.0
        cnt_acc[...] = jnp.zeros_like(cnt_acc)

    x = x_ref[...]
    z = jnp.dot(x, ws_ref[...],
                preferred_element_type=jnp.float32) + bs_ref[...]
    scores = sqr_acc[...] + jnp.dot(
        z, cbt2_acc[...], preferred_element_type=jnp.float32)
    minv = jnp.min(scores, axis=-1, keepdims=True)
    mask = scores == minv
    codes_f = jnp.min(jnp.where(mask, ids_acc[...], KF), axis=-1)
    codes_ref[0, 0, :] = codes_f.astype(jnp.int32)
    loss_acc[0, 0] += jnp.sum(minv) + jnp.sum(z * z)
    maskf = jnp.where(mask, 1.0, 0.0)
    cnt_acc[...] += jnp.sum(maskf, axis=0, keepdims=True)

    @pl.when(i == nsteps - 1)
    def _fin():
        loss_ref[0, 0] = loss_acc[0, 0]
        cnt_ref[...] = cnt_acc[...]


def _fin_kernel(l_ref, c_ref, loss_ref, ent_ref):
    loss_ref[0, 0] = jnp.sum(l_ref[...])
    cnt = jnp.sum(c_ref[...], axis=0, keepdims=True)
    p = cnt * (1.0 / N)
    plogp = jnp.where(p > 0.0, p * jnp.log(p), 0.0)
    ent_ref[0, 0] = -jnp.sum(plogp) * (1.0 / LN2)


def _chunk_call(c, x2, W_send, b_send, cbT):
    nsteps = CHUNKS[c] // T
    step0 = OFFSETS[c] // T
    return pl.pallas_call(
        functools.partial(_chunk_kernel, nsteps),
        grid=(nsteps,),
        in_specs=[
            pl.BlockSpec((T, C), lambda i, s=step0: (s + i, 0)),
            pl.BlockSpec((C, D), lambda i: (0, 0)),
            pl.BlockSpec((1, D), lambda i: (0, 0)),
            pl.BlockSpec((D, K), lambda i: (0, 0)),
        ],
        out_specs=[
            pl.BlockSpec((1, 1, T), lambda i: (i, 0, 0)),
            pl.BlockSpec(memory_space=pltpu.SMEM),
            pl.BlockSpec((1, K), lambda i: (0, 0)),
        ],
        out_shape=[
            jax.ShapeDtypeStruct((nsteps, 1, T), jnp.int32),
            jax.ShapeDtypeStruct((1, 1), jnp.float32),
            jax.ShapeDtypeStruct((1, K), jnp.float32),
        ],
        scratch_shapes=[
            pltpu.VMEM((1, K), jnp.float32),
            pltpu.VMEM((D, K), jnp.float32),
            pltpu.VMEM((1, K), jnp.float32),
            pltpu.SMEM((1, 1), jnp.float32),
            pltpu.VMEM((1, K), jnp.float32),
        ],
    )(x2, W_send, b_send, cbT)


GW = 128


def _sc_gather(table, idx2, out_rows, row0=0):
    nidx = idx2.shape[1]
    blk0 = row0 // GW
    mesh = plsc.VectorSubcoreMesh(core_axis_name="core",
                                  subcore_axis_name="subcore")

    @functools.partial(
        pl.kernel,
        out_type=jax.ShapeDtypeStruct((out_rows, C), jnp.float32),
        mesh=mesh,
    )
    def k(tab_hbm, idx_hbm, out_hbm):
        def body(idx_vmem, out_vmem):
            pltpu.sync_copy(tab_hbm.at[idx_vmem.at[0]], out_vmem)

        pltpu.emit_pipeline(
            body,
            grid=(nidx // GW,),
            in_specs=[pl.BlockSpec((1, GW), index_map=lambda i: (0, i))],
            out_specs=[pl.BlockSpec((GW, C),
                                    index_map=lambda i: (blk0 + i, 0))],
            core_axis_name=("core", "subcore"),
            dimension_semantics=(pltpu.PARALLEL,),
        )(idx_hbm, out_hbm)

    return k(table, idx2)


def kernel(input, W_send, b_send, codebook, W_recv, b_recv):
    x2 = input.reshape(N, C)
    cbT = codebook.T

    table = pl.pallas_call(
        _table_kernel,
        out_shape=jax.ShapeDtypeStruct((K, C), jnp.float32),
    )(codebook, W_recv, b_recv.reshape(1, C))

    codes_l, loss_l, cnt_l, x_l = [], [], [], []
    for c in range(CH):
        codes3, loss_c, cnt_c = _chunk_call(
            c, x2, W_send, b_send.reshape(1, D), cbT)
        codes_l.append(codes3.reshape(1, CHUNKS[c]))
        loss_l.append(loss_c)
        cnt_l.append(cnt_c)
        if c == FULL:
            x_l.append(_sc_gather(table, codes_l[c], N, row0=OFFSETS[c]))
        else:
            x_l.append(_sc_gather(table, codes_l[c], CHUNKS[c]))

    x = x_l[FULL]
    for c in range(CH):
        if c != FULL:
            x = lax.dynamic_update_slice(x, x_l[c], (OFFSETS[c], 0))

    loss, ent = pl.pallas_call(
        _fin_kernel,
        in_specs=[
            pl.BlockSpec((1, CH), lambda: (0, 0)),
            pl.BlockSpec((CH, K), lambda: (0, 0)),
        ],
        out_specs=[
            pl.BlockSpec(memory_space=pltpu.SMEM),
            pl.BlockSpec(memory_space=pltpu.SMEM),
        ],
        out_shape=[
            jax.ShapeDtypeStruct((1, 1), jnp.float32),
            jax.ShapeDtypeStruct((1, 1), jnp.float32),
        ],
    )(jnp.concatenate(loss_l, axis=1),
      jnp.concatenate(cnt_l, axis=0))

    codes = jnp.concatenate(codes_l, axis=1).reshape(64, 576)
    loss0 = loss.reshape(())
    return (x.reshape(64, 576, C), codes, loss0, loss0, ent.reshape(()))


# speedup vs baseline: 1.0974x; 1.0047x over previous
import functools

import jax
import jax.numpy as jnp
from jax import lax
from jax.experimental import pallas as pl
from jax.experimental.pallas import tpu as pltpu
from jax.experimental.pallas import tpu_sc as plsc

K = 1024
D = 64
DA = 72
C = 256
N = 64 * 576
T = 4096
LANES = 128

CHUNKS = (16384, 12288, 8192)
OFFSETS = (0, 16384, 28672)
FULL = 0
CH = len(CHUNKS)

LN2 = 0.6931471805599453
KF = float(K)


def _table_kernel(cb_ref, wr_ref, br_ref, tab_ref):
    tab_ref[...] = jnp.dot(cb_ref[...], wr_ref[...],
                           preferred_element_type=jnp.float32) + br_ref[...]


def _chunk_kernel(nsteps, x_ref, ws_ref, bs_ref, cbt_ref,
                  codes_ref, loss_ref, cnt_ref,
                  sqr_acc, cbt2_acc, ids_acc, loss_acc, cnt_acc):
    i = pl.program_id(0)

    @pl.when(i == 0)
    def _init():
        cbt = cbt_ref[...]
        sqr_acc[...] = jnp.sum(cbt * cbt, axis=0, keepdims=True)
        cbt2_acc[...] = cbt * -2.0
        ids_acc[...] = lax.broadcasted_iota(
            jnp.int32, (1, K), 1).astype(jnp.float32)
        loss_acc[0, 0] = 0.0
        cnt_acc[...] = jnp.zeros_like(cnt_acc)

    x = x_ref[...]
    z = jnp.dot(x, ws_ref[...],
                preferred_element_type=jnp.float32) + bs_ref[...]
    scores = sqr_acc[...] + jnp.dot(
        z, cbt2_acc[...], preferred_element_type=jnp.float32)
    minv = jnp.min(scores, axis=-1, keepdims=True)
    mask = scores == minv
    codes_f = jnp.min(jnp.where(mask, ids_acc[...], KF), axis=-1)
    codes_ref[0, 0, :] = codes_f.astype(jnp.int32)
    loss_acc[0, 0] += jnp.sum(minv) + jnp.sum(z * z)
    maskf = jnp.where(mask, 1.0, 0.0)
    cnt_acc[...] += jnp.sum(maskf, axis=0, keepdims=True)

    @pl.when(i == nsteps - 1)
    def _fin():
        loss_ref[0, 0] = loss_acc[0, 0]
        cnt_ref[...] = cnt_acc[...]


def _fin_kernel(l_ref, c_ref, loss_ref, ent_ref):
    loss_ref[0, 0] = jnp.sum(l_ref[...])
    cnt = jnp.sum(c_ref[...], axis=0, keepdims=True)
    p = cnt * (1.0 / N)
    plogp = jnp.where(p > 0.0, p * jnp.log(p), 0.0)
    ent_ref[0, 0] = -jnp.sum(plogp) * (1.0 / LN2)


def _chunk_call(c, x2, W_send, b_send, cbT):
    nsteps = CHUNKS[c] // T
    step0 = OFFSETS[c] // T
    return pl.pallas_call(
        functools.partial(_chunk_kernel, nsteps),
        grid=(nsteps,),
        in_specs=[
            pl.BlockSpec((T, C), lambda i, s=step0: (s + i, 0)),
            pl.BlockSpec((C, D), lambda i: (0, 0)),
            pl.BlockSpec((1, D), lambda i: (0, 0)),
            pl.BlockSpec((D, K), lambda i: (0, 0)),
        ],
        out_specs=[
            pl.BlockSpec((1, 1, T), lambda i: (i, 0, 0)),
            pl.BlockSpec(memory_space=pltpu.SMEM),
            pl.BlockSpec((1, K), lambda i: (0, 0)),
        ],
        out_shape=[
            jax.ShapeDtypeStruct((nsteps, 1, T), jnp.int32),
            jax.ShapeDtypeStruct((1, 1), jnp.float32),
            jax.ShapeDtypeStruct((1, K), jnp.float32),
        ],
        scratch_shapes=[
            pltpu.VMEM((1, K), jnp.float32),
            pltpu.VMEM((D, K), jnp.float32),
            pltpu.VMEM((1, K), jnp.float32),
            pltpu.SMEM((1, 1), jnp.float32),
            pltpu.VMEM((1, K), jnp.float32),
        ],
    )(x2, W_send, b_send, cbT)


GW = 128


def _sc_gather(table, idx2, out_rows, row0=0):
    nidx = idx2.shape[1]
    blk0 = row0 // GW
    mesh = plsc.VectorSubcoreMesh(core_axis_name="core",
                                  subcore_axis_name="subcore")

    @functools.partial(
        pl.kernel,
        out_type=jax.ShapeDtypeStruct((out_rows, C), jnp.float32),
        mesh=mesh,
    )
    def k(tab_hbm, idx_hbm, out_hbm):
        def body(idx_vmem, out_vmem):
            pltpu.sync_copy(tab_hbm.at[idx_vmem.at[0]], out_vmem)

        pltpu.emit_pipeline(
            body,
            grid=(nidx // GW,),
            in_specs=[pl.BlockSpec((1, GW), index_map=lambda i: (0, i))],
            out_specs=[pl.BlockSpec((GW, C),
                                    index_map=lambda i: (blk0 + i, 0))],
            core_axis_name=("core", "subcore"),
            dimension_semantics=(pltpu.PARALLEL,),
        )(idx_hbm, out_hbm)

    return k(table, idx2)


def kernel(input, W_send, b_send, codebook, W_recv, b_recv):
    x2 = input.reshape(N, C)
    cbT = codebook.T

    table = pl.pallas_call(
        _table_kernel,
        out_shape=jax.ShapeDtypeStruct((K, C), jnp.float32),
    )(codebook, W_recv, b_recv.reshape(1, C))

    codes_l, loss_l, cnt_l, x_l = [], [], [], []
    for c in range(CH):
        codes3, loss_c, cnt_c = _chunk_call(
            c, x2, W_send, b_send.reshape(1, D), cbT)
        codes_l.append(codes3.reshape(1, CHUNKS[c]))
        loss_l.append(loss_c)
        cnt_l.append(cnt_c)
        if c == FULL:
            x_l.append(_sc_gather(table, codes_l[c], N, row0=OFFSETS[c]))
        else:
            x_l.append(_sc_gather(table, codes_l[c], CHUNKS[c]))

    x = x_l[FULL]
    for c in range(CH):
        if c != FULL:
            x = lax.dynamic_update_slice(x, x_l[c], (OFFSETS[c], 0))

    loss, ent = pl.pallas_call(
        _fin_kernel,
        in_specs=[
            pl.BlockSpec((1, CH), lambda: (0, 0)),
            pl.BlockSpec((CH, K), lambda: (0, 0)),
        ],
        out_specs=[
            pl.BlockSpec(memory_space=pltpu.SMEM),
            pl.BlockSpec(memory_space=pltpu.SMEM),
        ],
        out_shape=[
            jax.ShapeDtypeStruct((1, 1), jnp.float32),
            jax.ShapeDtypeStruct((1, 1), jnp.float32),
        ],
    )(jnp.concatenate(loss_l, axis=1),
      jnp.concatenate(cnt_l, axis=0))

    codes = jnp.concatenate(codes_l, axis=1).reshape(64, 576)
    loss0 = loss.reshape(())
    return (x.reshape(64, 576, C), codes, loss0, loss0, ent.reshape(()))
